# speedup vs baseline: 18.6896x; 18.6896x over previous
import jax
import jax.numpy as jnp
from jax.experimental import pallas as pl
from jax.experimental.pallas import tpu as pltpu

B, C, H, W = 16, 13, 256, 256
MAXF = 64
N = H * W
BN = 4096
NB = N // BN


def _field_loss_kernel(lg_ref, fid_ref, lbl_ref, out_ref,
                       sums, hist, tacc, nacc, sacc):
    b = pl.program_id(0)
    j = pl.program_id(1)

    @pl.when((b == 0) & (j == 0))
    def _init_global():
        tacc[...] = jnp.zeros_like(tacc)
        nacc[...] = jnp.zeros_like(nacc)
        sacc[...] = jnp.zeros_like(sacc)

    @pl.when(j == 0)
    def _init_image():
        sums[...] = jnp.zeros_like(sums)
        hist[...] = jnp.zeros_like(hist)

    lg = lg_ref[0]
    fid = fid_ref[0]
    lbl = lbl_ref[0]

    oh_f = (jax.lax.broadcasted_iota(jnp.int32, (MAXF, BN), 0) == fid
            ).astype(jnp.float32)
    oh_l = (jax.lax.broadcasted_iota(jnp.int32, (C, BN), 0) == lbl
            ).astype(jnp.float32)

    dn = (((1,), (1,)), ((), ()))
    sums[...] += jax.lax.dot_general(oh_f, lg, dn,
                                     preferred_element_type=jnp.float32)
    hist[...] += jax.lax.dot_general(oh_f, oh_l, dn,
                                     preferred_element_type=jnp.float32)

    @pl.when(j == NB - 1)
    def _finish_image():
        s = sums[...]
        h = hist[...]
        counts = jnp.sum(h, axis=1, keepdims=True)
        mean = s / jnp.maximum(counts, 1.0)
        col = jax.lax.broadcasted_iota(jnp.int32, (MAXF, C), 1)
        vh = jnp.where(col == 0, 0.0, h)
        has_valid = jnp.sum(vh, axis=1, keepdims=True) > 0.0
        hh = jnp.where(has_valid, vh, h)
        m = jnp.max(hh, axis=1, keepdims=True)
        cand = jnp.where(hh == m, col, C)
        label = jnp.min(cand, axis=1, keepdims=True)
        mx = jnp.max(mean, axis=1, keepdims=True)
        lse = jnp.log(jnp.sum(jnp.exp(mean - mx), axis=1, keepdims=True)) + mx
        sel = jnp.sum(jnp.where(col == label, mean, 0.0), axis=1, keepdims=True)
        ce = lse - sel
        fidx = jax.lax.broadcasted_iota(jnp.int32, (MAXF, 1), 0)
        valid = ((counts > 0.0) & (fidx != 0)).astype(jnp.float32)
        tacc[...] += jnp.sum(ce * valid).reshape(1, 1)
        nacc[...] += jnp.sum(valid).reshape(1, 1)
        sacc[...] += jnp.sum(s).reshape(1, 1)

    @pl.when((b == B - 1) & (j == NB - 1))
    def _finish():
        t = tacc[...]
        n = nacc[...]
        s_all = sacc[...]
        out_ref[...] = jnp.where(n > 0.0, t / jnp.maximum(n, 1.0),
                                 s_all * 0.0)


def kernel(logits, masks, field_ids):
    lg = logits.reshape(B, C, N)
    fid = field_ids.reshape(B, 1, N)
    lbl = masks.reshape(B, 1, N)
    out = pl.pallas_call(
        _field_loss_kernel,
        grid=(B, NB),
        in_specs=[
            pl.BlockSpec((1, C, BN), lambda b, j: (b, 0, j)),
            pl.BlockSpec((1, 1, BN), lambda b, j: (b, 0, j)),
            pl.BlockSpec((1, 1, BN), lambda b, j: (b, 0, j)),
        ],
        out_specs=pl.BlockSpec((1, 1), lambda b, j: (0, 0)),
        out_shape=jax.ShapeDtypeStruct((1, 1), jnp.float32),
        scratch_shapes=[
            pltpu.VMEM((MAXF, C), jnp.float32),
            pltpu.VMEM((MAXF, C), jnp.float32),
            pltpu.VMEM((1, 1), jnp.float32),
            pltpu.VMEM((1, 1), jnp.float32),
            pltpu.VMEM((1, 1), jnp.float32),
        ],
    )(lg, fid, lbl)
    return out[0, 0]


# speedup vs baseline: 25.1031x; 1.3432x over previous
import jax
import jax.numpy as jnp
from jax.experimental import pallas as pl
from jax.experimental.pallas import tpu as pltpu

B, C, H, W = 16, 13, 256, 256
MAXF = 64
RB = 16
BN = RB * W
NB = H // RB


def _seg_kernel(lg_ref, fid_ref, lbl_ref, sums_ref, hist_ref):
    j = pl.program_id(1)

    @pl.when(j == 0)
    def _init():
        sums_ref[...] = jnp.zeros_like(sums_ref)
        hist_ref[...] = jnp.zeros_like(hist_ref)

    lg = lg_ref[0].reshape(C, BN)
    fid = fid_ref[0].reshape(1, BN)
    lbl = lbl_ref[0].reshape(1, BN)

    oh_f = (jax.lax.broadcasted_iota(jnp.int32, (MAXF, BN), 0) == fid
            ).astype(jnp.float32)
    oh_l = (jax.lax.broadcasted_iota(jnp.int32, (C, BN), 0) == lbl
            ).astype(jnp.float32)

    dn = (((1,), (1,)), ((), ()))
    sums_ref[0] += jax.lax.dot_general(oh_f, lg, dn,
                                       preferred_element_type=jnp.float32)
    hist_ref[0] += jax.lax.dot_general(oh_f, oh_l, dn,
                                       preferred_element_type=jnp.float32)


def _ce_kernel(s_ref, h_ref, out_ref):
    s = s_ref[...]
    h = h_ref[...]
    R = B * MAXF
    counts = jnp.sum(h, axis=1, keepdims=True)
    mean = s / jnp.maximum(counts, 1.0)
    col = jax.lax.broadcasted_iota(jnp.int32, (R, C), 1)
    vh = jnp.where(col == 0, 0.0, h)
    has_valid = jnp.sum(vh, axis=1, keepdims=True) > 0.0
    hh = jnp.where(has_valid, vh, h)
    m = jnp.max(hh, axis=1, keepdims=True)
    label = jnp.min(jnp.where(hh == m, col, C), axis=1, keepdims=True)
    mx = jnp.max(mean, axis=1, keepdims=True)
    lse = jnp.log(jnp.sum(jnp.exp(mean - mx), axis=1, keepdims=True)) + mx
    sel = jnp.sum(jnp.where(col == label, mean, 0.0), axis=1, keepdims=True)
    ce = lse - sel
    fidx = jax.lax.broadcasted_iota(jnp.int32, (R, 1), 0)
    valid = ((counts > 0.0) & ((fidx & (MAXF - 1)) != 0)).astype(jnp.float32)
    t = jnp.sum(ce * valid).reshape(1, 1)
    n = jnp.sum(valid).reshape(1, 1)
    sa = jnp.sum(s).reshape(1, 1)
    out_ref[...] = jnp.where(n > 0.0, t / jnp.maximum(n, 1.0), sa * 0.0)


def kernel(logits, masks, field_ids):
    sums, hist = pl.pallas_call(
        _seg_kernel,
        grid=(B, NB),
        in_specs=[
            pl.BlockSpec((1, C, RB, W), lambda b, j: (b, 0, j, 0)),
            pl.BlockSpec((1, RB, W), lambda b, j: (b, j, 0)),
            pl.BlockSpec((1, RB, W), lambda b, j: (b, j, 0)),
        ],
        out_specs=[
            pl.BlockSpec((1, MAXF, C), lambda b, j: (b, 0, 0)),
            pl.BlockSpec((1, MAXF, C), lambda b, j: (b, 0, 0)),
        ],
        out_shape=[
            jax.ShapeDtypeStruct((B, MAXF, C), jnp.float32),
            jax.ShapeDtypeStruct((B, MAXF, C), jnp.float32),
        ],
    )(logits, field_ids, masks)

    out = pl.pallas_call(
        _ce_kernel,
        out_shape=jax.ShapeDtypeStruct((1, 1), jnp.float32),
    )(sums.reshape(B * MAXF, C), hist.reshape(B * MAXF, C))
    return out[0, 0]


# speedup vs baseline: 54.1035x; 2.1553x over previous
import jax
import jax.numpy as jnp
from jax.experimental import pallas as pl
from jax.experimental.pallas import tpu as pltpu

B, C, H, W = 16, 13, 256, 256
MAXF = 64
RB = 64
BN = RB * W
NB = H // RB


def _seg_kernel(lg_ref, fid_ref, lbl_ref, sums_ref, hist_ref):
    j = pl.program_id(1)

    @pl.when(j == 0)
    def _init():
        sums_ref[...] = jnp.zeros_like(sums_ref)
        hist_ref[...] = jnp.zeros_like(hist_ref)

    lg = lg_ref[0].reshape(C, BN)
    fid = fid_ref[0].reshape(1, BN)
    lbl = lbl_ref[0].reshape(1, BN)

    oh_f = (jax.lax.broadcasted_iota(jnp.int32, (MAXF, BN), 0) == fid
            ).astype(jnp.float32)
    oh_l = (jax.lax.broadcasted_iota(jnp.int32, (C, BN), 0) == lbl
            ).astype(jnp.float32)

    dn = (((1,), (1,)), ((), ()))
    sums_ref[0] += jax.lax.dot_general(oh_f, lg, dn,
                                       preferred_element_type=jnp.float32)
    hist_ref[0] += jax.lax.dot_general(oh_f, oh_l, dn,
                                       preferred_element_type=jnp.float32)


def _ce_kernel(s_ref, h_ref, out_ref):
    s = s_ref[...]
    h = h_ref[...]
    R = B * MAXF
    counts = jnp.sum(h, axis=1, keepdims=True)
    mean = s / jnp.maximum(counts, 1.0)
    col = jax.lax.broadcasted_iota(jnp.int32, (R, C), 1)
    vh = jnp.where(col == 0, 0.0, h)
    has_valid = jnp.sum(vh, axis=1, keepdims=True) > 0.0
    hh = jnp.where(has_valid, vh, h)
    m = jnp.max(hh, axis=1, keepdims=True)
    label = jnp.min(jnp.where(hh == m, col, C), axis=1, keepdims=True)
    mx = jnp.max(mean, axis=1, keepdims=True)
    lse = jnp.log(jnp.sum(jnp.exp(mean - mx), axis=1, keepdims=True)) + mx
    sel = jnp.sum(jnp.where(col == label, mean, 0.0), axis=1, keepdims=True)
    ce = lse - sel
    fidx = jax.lax.broadcasted_iota(jnp.int32, (R, 1), 0)
    valid = ((counts > 0.0) & ((fidx & (MAXF - 1)) != 0)).astype(jnp.float32)
    t = jnp.sum(ce * valid).reshape(1, 1)
    n = jnp.sum(valid).reshape(1, 1)
    sa = jnp.sum(s).reshape(1, 1)
    out_ref[...] = jnp.where(n > 0.0, t / jnp.maximum(n, 1.0), sa * 0.0)


def kernel(logits, masks, field_ids):
    sums, hist = pl.pallas_call(
        _seg_kernel,
        grid=(B, NB),
        in_specs=[
            pl.BlockSpec((1, C, RB, W), lambda b, j: (b, 0, j, 0)),
            pl.BlockSpec((1, RB, W), lambda b, j: (b, j, 0)),
            pl.BlockSpec((1, RB, W), lambda b, j: (b, j, 0)),
        ],
        out_specs=[
            pl.BlockSpec((1, MAXF, C), lambda b, j: (b, 0, 0)),
            pl.BlockSpec((1, MAXF, C), lambda b, j: (b, 0, 0)),
        ],
        out_shape=[
            jax.ShapeDtypeStruct((B, MAXF, C), jnp.float32),
            jax.ShapeDtypeStruct((B, MAXF, C), jnp.float32),
        ],
    )(logits, field_ids, masks)

    out = pl.pallas_call(
        _ce_kernel,
        out_shape=jax.ShapeDtypeStruct((1, 1), jnp.float32),
    )(sums.reshape(B * MAXF, C), hist.reshape(B * MAXF, C))
    return out[0, 0]


# speedup vs baseline: 58.6081x; 1.0833x over previous
import jax
import jax.numpy as jnp
from jax.experimental import pallas as pl
from jax.experimental.pallas import tpu as pltpu

B, C, H, W = 16, 13, 256, 256
MAXF = 64
RB = 128
BN = RB * W
NB = H // RB


def _seg_kernel(lg_ref, fid_ref, lbl_ref, sums_ref, hist_ref):
    j = pl.program_id(1)

    @pl.when(j == 0)
    def _init():
        sums_ref[...] = jnp.zeros_like(sums_ref)
        hist_ref[...] = jnp.zeros_like(hist_ref)

    lg = lg_ref[0].reshape(C, BN)
    fid = fid_ref[0].reshape(1, BN)
    lbl = lbl_ref[0].reshape(1, BN)

    oh_f = (jax.lax.broadcasted_iota(jnp.int32, (MAXF, BN), 0) == fid
            ).astype(jnp.float32)
    oh_l = (jax.lax.broadcasted_iota(jnp.int32, (C, BN), 0) == lbl
            ).astype(jnp.float32)

    dn = (((1,), (1,)), ((), ()))
    sums_ref[0] += jax.lax.dot_general(oh_f, lg, dn,
                                       preferred_element_type=jnp.float32)
    hist_ref[0] += jax.lax.dot_general(oh_f, oh_l, dn,
                                       preferred_element_type=jnp.float32)


def _ce_kernel(s_ref, h_ref, out_ref):
    s = s_ref[...]
    h = h_ref[...]
    R = B * MAXF
    counts = jnp.sum(h, axis=1, keepdims=True)
    mean = s / jnp.maximum(counts, 1.0)
    col = jax.lax.broadcasted_iota(jnp.int32, (R, C), 1)
    vh = jnp.where(col == 0, 0.0, h)
    has_valid = jnp.sum(vh, axis=1, keepdims=True) > 0.0
    hh = jnp.where(has_valid, vh, h)
    m = jnp.max(hh, axis=1, keepdims=True)
    label = jnp.min(jnp.where(hh == m, col, C), axis=1, keepdims=True)
    mx = jnp.max(mean, axis=1, keepdims=True)
    lse = jnp.log(jnp.sum(jnp.exp(mean - mx), axis=1, keepdims=True)) + mx
    sel = jnp.sum(jnp.where(col == label, mean, 0.0), axis=1, keepdims=True)
    ce = lse - sel
    fidx = jax.lax.broadcasted_iota(jnp.int32, (R, 1), 0)
    valid = ((counts > 0.0) & ((fidx & (MAXF - 1)) != 0)).astype(jnp.float32)
    t = jnp.sum(ce * valid).reshape(1, 1)
    n = jnp.sum(valid).reshape(1, 1)
    sa = jnp.sum(s).reshape(1, 1)
    out_ref[...] = jnp.where(n > 0.0, t / jnp.maximum(n, 1.0), sa * 0.0)


def kernel(logits, masks, field_ids):
    sums, hist = pl.pallas_call(
        _seg_kernel,
        grid=(B, NB),
        in_specs=[
            pl.BlockSpec((1, C, RB, W), lambda b, j: (b, 0, j, 0)),
            pl.BlockSpec((1, RB, W), lambda b, j: (b, j, 0)),
            pl.BlockSpec((1, RB, W), lambda b, j: (b, j, 0)),
        ],
        out_specs=[
            pl.BlockSpec((1, MAXF, C), lambda b, j: (b, 0, 0)),
            pl.BlockSpec((1, MAXF, C), lambda b, j: (b, 0, 0)),
        ],
        out_shape=[
            jax.ShapeDtypeStruct((B, MAXF, C), jnp.float32),
            jax.ShapeDtypeStruct((B, MAXF, C), jnp.float32),
        ],
    )(logits, field_ids, masks)

    out = pl.pallas_call(
        _ce_kernel,
        out_shape=jax.ShapeDtypeStruct((1, 1), jnp.float32),
    )(sums.reshape(B * MAXF, C), hist.reshape(B * MAXF, C))
    return out[0, 0]


# speedup vs baseline: 89.2660x; 1.5231x over previous
import jax
import jax.numpy as jnp
from jax.experimental import pallas as pl
from jax.experimental.pallas import tpu as pltpu

B, C, H, W = 16, 13, 256, 256
MAXF = 64
RB = 128
BN = RB * W
NB = H // RB


def _seg_kernel(lg_ref, fid_ref, lbl_ref, sums_ref, hist_ref):
    j = pl.program_id(1)

    @pl.when(j == 0)
    def _init():
        sums_ref[...] = jnp.zeros_like(sums_ref)
        hist_ref[...] = jnp.zeros_like(hist_ref)

    lg = lg_ref[0].reshape(C, BN)
    fid = fid_ref[0].reshape(1, BN)
    lbl = lbl_ref[0].reshape(1, BN)

    oh_f = (jax.lax.broadcasted_iota(jnp.int32, (MAXF, BN), 0) == fid
            ).astype(jnp.float32)
    oh_l = (jax.lax.broadcasted_iota(jnp.int32, (C, BN), 0) == lbl
            ).astype(jnp.float32)

    dn = (((1,), (1,)), ((), ()))
    rhs = jnp.concatenate([lg, oh_l], axis=0)
    both = jax.lax.dot_general(oh_f, rhs, dn,
                               preferred_element_type=jnp.float32)
    sums_ref[0] += both[:, :C]
    hist_ref[0] += both[:, C:]


def _ce_kernel(s_ref, h_ref, out_ref):
    s = s_ref[...]
    h = h_ref[...]
    R = B * MAXF
    counts = jnp.sum(h, axis=1, keepdims=True)
    mean = s / jnp.maximum(counts, 1.0)
    col = jax.lax.broadcasted_iota(jnp.int32, (R, C), 1)
    vh = jnp.where(col == 0, 0.0, h)
    has_valid = jnp.sum(vh, axis=1, keepdims=True) > 0.0
    hh = jnp.where(has_valid, vh, h)
    m = jnp.max(hh, axis=1, keepdims=True)
    label = jnp.min(jnp.where(hh == m, col, C), axis=1, keepdims=True)
    mx = jnp.max(mean, axis=1, keepdims=True)
    lse = jnp.log(jnp.sum(jnp.exp(mean - mx), axis=1, keepdims=True)) + mx
    sel = jnp.sum(jnp.where(col == label, mean, 0.0), axis=1, keepdims=True)
    ce = lse - sel
    fidx = jax.lax.broadcasted_iota(jnp.int32, (R, 1), 0)
    valid = ((counts > 0.0) & ((fidx & (MAXF - 1)) != 0)).astype(jnp.float32)
    t = jnp.sum(ce * valid).reshape(1, 1)
    n = jnp.sum(valid).reshape(1, 1)
    sa = jnp.sum(s).reshape(1, 1)
    out_ref[...] = jnp.where(n > 0.0, t / jnp.maximum(n, 1.0), sa * 0.0)


def kernel(logits, masks, field_ids):
    sums, hist = pl.pallas_call(
        _seg_kernel,
        grid=(B, NB),
        in_specs=[
            pl.BlockSpec((1, C, RB, W), lambda b, j: (b, 0, j, 0)),
            pl.BlockSpec((1, RB, W), lambda b, j: (b, j, 0)),
            pl.BlockSpec((1, RB, W), lambda b, j: (b, j, 0)),
        ],
        out_specs=[
            pl.BlockSpec((1, MAXF, C), lambda b, j: (b, 0, 0)),
            pl.BlockSpec((1, MAXF, C), lambda b, j: (b, 0, 0)),
        ],
        out_shape=[
            jax.ShapeDtypeStruct((B, MAXF, C), jnp.float32),
            jax.ShapeDtypeStruct((B, MAXF, C), jnp.float32),
        ],
    )(logits, field_ids, masks)

    out = pl.pallas_call(
        _ce_kernel,
        out_shape=jax.ShapeDtypeStruct((1, 1), jnp.float32),
    )(sums.reshape(B * MAXF, C), hist.reshape(B * MAXF, C))
    return out[0, 0]
